# baseline (speedup 1.0000x reference)
import numpy as np
import jax
import jax.numpy as jnp
from jax import lax
from jax.experimental import pallas as pl
from jax.experimental.pallas import tpu as pltpu
from jax.experimental.pallas import tpu_sc as plsc

_B = 256
_LMAX = 256
_M = 2 * _LMAX - 1
_K = _LMAX * _LMAX
_NC, _NS, _L = 2, 16, 16
_NW = _NC * _NS
_KS = _K // _NW
_G = _KS // _L

_NR_BUCKETS = (64, 96, 128, 176, 216, 256)


def _build_geometry():
    cols = np.arange(_M)
    starts = np.abs(cols - (_LMAX - 1))
    l_of_k = np.concatenate([np.arange(s, _LMAX) for s in starts])
    c_of_k = np.repeat(cols, _LMAX - starts)

    raw = []
    for s in range(_NW):
        sl = slice(s * _KS, (s + 1) * _KS)
        lk, ck = l_of_k[sl], c_of_k[sl]
        raw.append((int(lk.min()), int(lk.max()), int(ck.min()), int(ck.max())))

    static_spans = [_NW - 1]

    cls_of_span = {}
    for s, (r0, r1, c0, c1) in enumerate(raw):
        if s in static_spans:
            continue
        nr = r1 - r0 + 1
        cls_of_span[s] = next(i for i, t in enumerate(_NR_BUCKETS) if nr <= t)
    classes = []
    for ci, nr_c in enumerate(_NR_BUCKETS):
        members = [s for s in sorted(cls_of_span) if cls_of_span[s] == ci]
        if not members:
            continue
        w_c = max(raw[s][3] - (raw[s][2] // 8) * 8 + 1 for s in members)
        w_c = min(-(-w_c // 8) * 8, _M)
        offs = []
        for s in members:
            r0, r1, c0, c1 = raw[s]
            r0c = min(r0, _LMAX - nr_c)
            c0c = min((c0 // 8) * 8, ((_M - w_c) // 8) * 8)
            assert r0c >= 0 and r0c + nr_c >= r1 + 1
            assert c0c >= 0 and c0c + w_c >= c1 + 1
            offs.append((s, r0c, c0c))
        classes.append((nr_c, w_c, offs))

    for s in static_spans:
        r0, r1, c0, c1 = raw[s]
        c0a = (c0 // 8) * 8
        classes.append((r1 - r0 + 1, c1 - c0a + 1, [(s, r0, c0a)]))

    packed = np.zeros((_NW, _KS), np.int32)
    for nr_c, w_c, offs in classes:
        for s, r0c, c0c in offs:
            sl = slice(s * _KS, (s + 1) * _KS)
            lk, ck = l_of_k[sl], c_of_k[sl]
            packed[s] = (((lk - r0c).astype(np.int32) << 16)
                         | (ck - c0c).astype(np.int32))
    return classes, packed


_CLASSES, _PACKED = _build_geometry()


def _sc_body(tensor_hbm, idx_hbm, out_hbm, idx_v, ob0, ob1, isem0, isem1,
             osem0, osem1):
    wid = lax.axis_index("c") * _NS + lax.axis_index("s")
    pltpu.sync_copy(idx_hbm.at[wid], idx_v)
    out_off = wid * _KS

    def gather(slab, ob):
        @pl.loop(0, _G, unroll=8)
        def _g(g):
            iv = idx_v[pl.ds(g * _L, _L)]
            rows = lax.shift_right_logical(iv, 16)
            cls_ = lax.bitwise_and(iv, jnp.int32(0xFFFF))
            ob[pl.ds(g * _L, _L)] = plsc.load_gather(slab, [rows, cls_])

    for nr_c, w_c, offs in _CLASSES:
        if len(offs) == 1:
            s, r0c, c0c = offs[0]
            is_member = wid == s
        else:
            is_member = jnp.bool_(False)
            r0c = jnp.int32(0)
            c0c = jnp.int32(0)
            for s, r0, c0 in offs:
                hit = wid == s
                is_member = jnp.logical_or(is_member, hit)
                r0c = jnp.where(hit, jnp.int32(r0), r0c)
                c0c = jnp.where(hit, jnp.int32(c0), c0c)
            c0c = pl.multiple_of(c0c, 8)

        pitch = w_c if w_c % 2 else w_c + 1

        @pl.when(is_member)
        def _cls(nr_c=nr_c, w_c=w_c, r0c=r0c, c0c=c0c, pitch=pitch):
            def scoped(slab0, slab1):
                def in_copy(b, slab, sem):
                    dst = slab if pitch == w_c else slab.at[:, pl.ds(0, w_c)]
                    return pltpu.make_async_copy(
                        tensor_hbm.at[b, pl.ds(r0c, nr_c), pl.ds(c0c, w_c)],
                        dst, sem)

                def out_copy(b, ob, sem):
                    return pltpu.make_async_copy(
                        ob, out_hbm.at[b, pl.ds(out_off, _KS)], sem)

                in_copy(0, slab0, isem0).start()
                in_copy(1, slab1, isem1).start()

                @pl.loop(0, _B // 2)
                def _bb(bb):
                    b0 = bb * 2
                    b1 = b0 + 1

                    @pl.when(bb > 0)
                    def _():
                        out_copy(b0 - 2, ob0, osem0).wait()
                    in_copy(b0, slab0, isem0).wait()
                    gather(slab0, ob0)

                    @pl.when(bb < _B // 2 - 1)
                    def _():
                        in_copy(b0 + 2, slab0, isem0).start()
                    out_copy(b0, ob0, osem0).start()

                    @pl.when(bb > 0)
                    def _():
                        out_copy(b1 - 2, ob1, osem1).wait()
                    in_copy(b1, slab1, isem1).wait()
                    gather(slab1, ob1)

                    @pl.when(bb < _B // 2 - 1)
                    def _():
                        in_copy(b1 + 2, slab1, isem1).start()
                    out_copy(b1, ob1, osem1).start()

                out_copy(_B - 2, ob0, osem0).wait()
                out_copy(_B - 1, ob1, osem1).wait()

            pl.run_scoped(
                scoped,
                pltpu.VMEM((nr_c, pitch), jnp.float32),
                pltpu.VMEM((nr_c, pitch), jnp.float32),
            )


def kernel(tensor):
    idx = jnp.asarray(_PACKED)
    mesh = plsc.VectorSubcoreMesh(core_axis_name="c", subcore_axis_name="s")
    f = pl.kernel(
        _sc_body,
        out_type=jax.ShapeDtypeStruct((_B, _K), jnp.float32),
        mesh=mesh,
        compiler_params=pltpu.CompilerParams(
            use_tc_tiling_on_sc=False, needs_layout_passes=False),
        scratch_types=[
            pltpu.VMEM((_KS,), jnp.int32),
            pltpu.VMEM((_KS,), jnp.float32),
            pltpu.VMEM((_KS,), jnp.float32),
            pltpu.SemaphoreType.DMA,
            pltpu.SemaphoreType.DMA,
            pltpu.SemaphoreType.DMA,
            pltpu.SemaphoreType.DMA,
        ],
    )
    return f(tensor, idx)


# speedup vs baseline: 2.2092x; 2.2092x over previous
import numpy as np
import jax
import jax.numpy as jnp
from jax import lax
from jax.experimental import pallas as pl
from jax.experimental.pallas import tpu as pltpu
from jax.experimental.pallas import tpu_sc as plsc

_B = 256
_LMAX = 256
_M = 2 * _LMAX - 1
_K = _LMAX * _LMAX
_NC, _NS, _L = 2, 16, 16
_NW = _NC * _NS
_KS = _K // _NW
_G = _KS // _L

_W_BUCKETS = (12, 16, 20, 24, 32, 64)


def _build_geometry():
    cols = np.arange(_M)
    starts = np.abs(cols - (_LMAX - 1))
    l_of_k = np.concatenate([np.arange(s, _LMAX) for s in starts])
    c_of_k = np.repeat(cols, _LMAX - starts)

    raw = []
    for s in range(_NW):
        sl = slice(s * _KS, (s + 1) * _KS)
        ck = c_of_k[sl]
        raw.append((int(ck.min()), int(ck.max())))

    cls_of_span = []
    for c0, c1 in raw:
        w = c1 - c0 + 1
        cls_of_span.append(next(i for i, t in enumerate(_W_BUCKETS) if w <= t))

    classes = []
    for ci, w_c in enumerate(_W_BUCKETS):
        members = [s for s in range(_NW) if cls_of_span[s] == ci]
        if not members:
            continue
        offs = []
        for s in members:
            c0, c1 = raw[s]
            c0c = min(c0, _M - w_c)
            assert c0c >= 0 and c0c + w_c >= c1 + 1
            offs.append((s, c0c))
        classes.append((w_c, offs))

    packed = np.zeros((_NW, _KS), np.int32)
    for w_c, offs in classes:
        for s, c0c in offs:
            sl = slice(s * _KS, (s + 1) * _KS)
            lk, ck = l_of_k[sl], c_of_k[sl]
            packed[s] = (((ck - c0c).astype(np.int32) << 16)
                         | lk.astype(np.int32))
    return classes, packed


_CLASSES, _PACKED = _build_geometry()


def _sc_body(tensor_hbm, idx_hbm, out_hbm, idx_v, ob0, ob1, isem0, isem1,
             osem0, osem1):
    wid = lax.axis_index("c") * _NS + lax.axis_index("s")
    pltpu.sync_copy(idx_hbm.at[wid], idx_v)
    out_off = wid * _KS

    def gather(slab, ob):
        @pl.loop(0, _G, unroll=8)
        def _g(g):
            iv = idx_v[pl.ds(g * _L, _L)]
            rows = lax.shift_right_logical(iv, 16)
            cls_ = lax.bitwise_and(iv, jnp.int32(0xFFFF))
            ob[pl.ds(g * _L, _L)] = plsc.load_gather(slab, [rows, cls_])

    for w_c, offs in _CLASSES:
        if len(offs) == 1:
            s, c0c = offs[0]
            is_member = wid == s
        else:
            is_member = jnp.bool_(False)
            c0c = jnp.int32(0)
            for s, c0 in offs:
                hit = wid == s
                is_member = jnp.logical_or(is_member, hit)
                c0c = jnp.where(hit, jnp.int32(c0), c0c)

        @pl.when(is_member)
        def _cls(w_c=w_c, c0c=c0c):
            def scoped(slab0, slab1):
                def in_copy(b, slab, sem):
                    return pltpu.make_async_copy(
                        tensor_hbm.at[b, pl.ds(c0c, w_c), :], slab, sem)

                def out_copy(b, ob, sem):
                    return pltpu.make_async_copy(
                        ob, out_hbm.at[b, pl.ds(out_off, _KS)], sem)

                in_copy(0, slab0, isem0).start()
                in_copy(1, slab1, isem1).start()

                @pl.loop(0, _B // 2)
                def _bb(bb):
                    b0 = bb * 2
                    b1 = b0 + 1

                    @pl.when(bb > 0)
                    def _():
                        out_copy(b0 - 2, ob0, osem0).wait()
                    in_copy(b0, slab0, isem0).wait()
                    gather(slab0, ob0)

                    @pl.when(bb < _B // 2 - 1)
                    def _():
                        in_copy(b0 + 2, slab0, isem0).start()
                    out_copy(b0, ob0, osem0).start()

                    @pl.when(bb > 0)
                    def _():
                        out_copy(b1 - 2, ob1, osem1).wait()
                    in_copy(b1, slab1, isem1).wait()
                    gather(slab1, ob1)

                    @pl.when(bb < _B // 2 - 1)
                    def _():
                        in_copy(b1 + 2, slab1, isem1).start()
                    out_copy(b1, ob1, osem1).start()

                out_copy(_B - 2, ob0, osem0).wait()
                out_copy(_B - 1, ob1, osem1).wait()

            pl.run_scoped(
                scoped,
                pltpu.VMEM((w_c, _LMAX), jnp.float32),
                pltpu.VMEM((w_c, _LMAX), jnp.float32),
            )


def kernel(tensor):
    tensor_t = jnp.swapaxes(tensor, 1, 2)
    idx = jnp.asarray(_PACKED)
    mesh = plsc.VectorSubcoreMesh(core_axis_name="c", subcore_axis_name="s")
    f = pl.kernel(
        _sc_body,
        out_type=jax.ShapeDtypeStruct((_B, _K), jnp.float32),
        mesh=mesh,
        compiler_params=pltpu.CompilerParams(
            use_tc_tiling_on_sc=False, needs_layout_passes=False),
        scratch_types=[
            pltpu.VMEM((_KS,), jnp.int32),
            pltpu.VMEM((_KS,), jnp.float32),
            pltpu.VMEM((_KS,), jnp.float32),
            pltpu.SemaphoreType.DMA,
            pltpu.SemaphoreType.DMA,
            pltpu.SemaphoreType.DMA,
            pltpu.SemaphoreType.DMA,
        ],
    )
    return f(tensor_t, idx)
